# baseline (speedup 1.0000x reference)
import functools

import jax
import jax.numpy as jnp
from jax import lax
from jax.experimental import pallas as pl
from jax.experimental.pallas import tpu as pltpu
from jax.experimental.pallas import tpu_sc as plsc

_S = 128
_B = 8
_FLAT = _S * _S
_PAD = 64
_INV_COUNT = 1.0 / (_B * (_S - 2))

_D0S = ((1, 17, 97, 113), (33, 49, 65, 81))
_PHASE_END = ((16, 32, 112, 128), (48, 64, 80, 96))


def _sqrt16(x):
    y = (x + 1.0) * 0.5
    for _ in range(12):
        y = 0.5 * (y + x / y)
    return y


def _make_kernel():
    mesh = plsc.VectorSubcoreMesh(
        core_axis_name="c", subcore_axis_name="s", num_cores=1
    )

    @functools.partial(
        pl.kernel,
        mesh=mesh,
        out_type=jax.ShapeDtypeStruct((16, 16), jnp.float32),
        compiler_params=pltpu.CompilerParams(needs_layout_passes=False),
        scratch_types=[
            pltpu.VMEM((_FLAT + _PAD,), jnp.float32),
            pltpu.VMEM((16,), jnp.float32),
            pltpu.SemaphoreType.DMA,
            pltpu.SemaphoreType.DMA,
            pltpu.SemaphoreType.DMA,
            pltpu.SemaphoreType.DMA,
        ],
    )
    def diag_std_kernel(attn_hbm, out_hbm, buf, part_v, s0, s1, s2, s3):
        s = lax.axis_index("s")
        batch = s >> 1
        half = s & 1
        sems = (s0, s1, s2, s3)
        lane = lax.iota(jnp.int32, 16)
        zero = jnp.zeros((16,), jnp.float32)

        def unit(base, dv, i, sacc, qacc):
            x = buf[pl.ds(base, 16)]
            m = (dv + i) < _S
            x = jnp.where(m, x, 0.0)
            return sacc + x, qacc + x * x

        def finalize(dv, sx, qx):
            nf = (_S - dv).astype(jnp.float32)
            var = (qx - sx * sx / nf) / (nf - 1.0)
            var = jnp.maximum(var, 0.0)
            std = _sqrt16(var)
            return jnp.where(dv <= _S - 2, std * nf * 0.2, 0.0)

        def make_branch(h):
            d0s = _D0S[h]
            ends = _PHASE_END[h]
            dvs = [d0 + lane for d0 in d0s]

            def branch():
                copies = []
                off = 0
                for p in range(4):
                    words = min(ends[p], _S - 1) * _S - off
                    copies.append(
                        pltpu.async_copy(
                            attn_hbm.at[batch, pl.ds(off, words)],
                            buf.at[pl.ds(off, words)],
                            sems[p],
                        )
                    )
                    off += words

                accs = [(zero, zero)] * 4
                row0 = 0
                for phase in range(4):
                    copies[phase].wait()
                    nchunks = 4 - phase
                    nb = (ends[phase] - row0) // 2

                    def body(j, carry, row0=row0, nchunks=nchunks):
                        out = list(carry)
                        i = row0 + j * 2
                        for k in range(nchunks):
                            sc_, qc_ = out[2 * k], out[2 * k + 1]
                            base = d0s[k] + 129 * i
                            sc_, qc_ = unit(base, dvs[k], i, sc_, qc_)
                            sc_, qc_ = unit(base + 129, dvs[k], i + 1, sc_, qc_)
                            out[2 * k], out[2 * k + 1] = sc_, qc_
                        return tuple(out)

                    flat = tuple(x for pair in accs[:nchunks] for x in pair)
                    flat = lax.fori_loop(0, nb, body, flat)
                    for k in range(nchunks):
                        accs[k] = (flat[2 * k], flat[2 * k + 1])
                    row0 = ends[phase]

                partial = zero
                for k in range(4):
                    partial = partial + finalize(dvs[k], accs[k][0], accs[k][1])
                part_v[...] = partial * _INV_COUNT
                pltpu.sync_copy(part_v, out_hbm.at[s])

            return branch

        lax.switch(half, [make_branch(0), make_branch(1)])

    return diag_std_kernel


_diag_std = _make_kernel()


def kernel(attn):
    flat = attn.reshape(_B, _FLAT)
    out = _diag_std(flat)
    return jnp.sum(out)


# speedup vs baseline: 1.0332x; 1.0332x over previous
import functools

import jax
import jax.numpy as jnp
from jax import lax
from jax.experimental import pallas as pl
from jax.experimental.pallas import tpu as pltpu
from jax.experimental.pallas import tpu_sc as plsc

_S = 128
_B = 8
_FLAT = _S * _S
_PAD = 64
_INV_COUNT = 1.0 / (_B * (_S - 2))


def _sqrt16(x):
    y = (x + 1.0) * 0.5
    for _ in range(12):
        y = 0.5 * (y + x / y)
    return y


def _make_kernel():
    mesh = plsc.VectorSubcoreMesh(
        core_axis_name="c", subcore_axis_name="s", num_cores=1
    )

    @functools.partial(
        pl.kernel,
        mesh=mesh,
        out_type=jax.ShapeDtypeStruct((16, 16), jnp.float32),
        compiler_params=pltpu.CompilerParams(needs_layout_passes=False),
        scratch_types=[
            pltpu.VMEM((_FLAT + _PAD,), jnp.float32),
            pltpu.VMEM((16,), jnp.float32),
        ],
    )
    def diag_std_kernel(attn_hbm, out_hbm, buf, part_v):
        s = lax.axis_index("s")
        batch = s >> 1
        half = s & 1

        def _copy(nrows):
            def f():
                pltpu.sync_copy(
                    attn_hbm.at[batch, pl.ds(0, nrows * _S)],
                    buf.at[pl.ds(0, nrows * _S)],
                )
            return f

        lax.switch(half, [_copy(127), _copy(95)])

        lane = lax.iota(jnp.int32, 16)

        def sel(a, b):
            return jnp.where(half == 0, a, b)

        d0s = [sel(1, 33), sel(17, 49), sel(97, 65), sel(113, 81)]
        dvs = [d0 + lane for d0 in d0s]
        nblocks = [sel(4, 12), 4, sel(20, 4), 4]

        def unit(base, dv, i, sacc, qacc):
            x = buf[pl.ds(base, 16)]
            m = (dv + i) < _S
            x = jnp.where(m, x, 0.0)
            return sacc + x, qacc + x * x

        zero = jnp.zeros((16,), jnp.float32)
        accs = [(zero, zero)] * 4
        row0 = 0
        for phase in range(4):
            nchunks = 4 - phase

            def body(j, carry, row0=row0, nchunks=nchunks):
                out = list(carry)
                i = row0 + j * 4
                for k in range(nchunks):
                    sc_, qc_ = out[2 * k], out[2 * k + 1]
                    base = d0s[k] + 129 * i
                    for u in range(4):
                        sc_, qc_ = unit(base + 129 * u, dvs[k], i + u, sc_, qc_)
                    out[2 * k], out[2 * k + 1] = sc_, qc_
                return tuple(out)

            flat_accs = tuple(x for pair in accs[:nchunks] for x in pair)
            flat_accs = lax.fori_loop(0, nblocks[phase], body, flat_accs)
            for k in range(nchunks):
                accs[k] = (flat_accs[2 * k], flat_accs[2 * k + 1])
            row0 = row0 + nblocks[phase] * 4

        def finalize(dv, sx, qx):
            nf = (_S - dv).astype(jnp.float32)
            var = (qx - sx * sx / nf) / (nf - 1.0)
            var = jnp.maximum(var, 0.0)
            std = _sqrt16(var)
            return jnp.where(dv <= _S - 2, std * nf * 0.2, 0.0)

        partial = zero
        for k in range(4):
            partial = partial + finalize(dvs[k], accs[k][0], accs[k][1])
        part_v[...] = partial * _INV_COUNT
        pltpu.sync_copy(part_v, out_hbm.at[s])

    return diag_std_kernel


_diag_std = _make_kernel()


def kernel(attn):
    flat = attn.reshape(_B, _FLAT)
    out = _diag_std(flat)
    return jnp.sum(out)


# speedup vs baseline: 1.0341x; 1.0009x over previous
import functools

import jax
import jax.numpy as jnp
from jax import lax
from jax.experimental import pallas as pl
from jax.experimental.pallas import tpu as pltpu
from jax.experimental.pallas import tpu_sc as plsc

_S = 128
_B = 8
_FLAT = _S * _S
_PAD = 64
_INV_COUNT = 1.0 / (_B * (_S - 2))


def _sqrt16(x):
    y = (x + 1.0) * 0.5
    for _ in range(12):
        y = 0.5 * (y + x / y)
    return y


def _make_kernel():
    mesh = plsc.VectorSubcoreMesh(
        core_axis_name="c", subcore_axis_name="s", num_cores=1
    )

    @functools.partial(
        pl.kernel,
        mesh=mesh,
        out_type=jax.ShapeDtypeStruct((16, 16), jnp.float32),
        compiler_params=pltpu.CompilerParams(needs_layout_passes=False),
        scratch_types=[
            pltpu.VMEM((_FLAT + _PAD,), jnp.float32),
            pltpu.VMEM((16,), jnp.float32),
            pltpu.SemaphoreType.DMA,
            pltpu.SemaphoreType.DMA,
        ],
    )
    def diag_std_kernel(attn_hbm, out_hbm, buf, part_v, sem1, sem2):
        s = lax.axis_index("s")
        batch = s >> 1
        half = s & 1

        copy1 = pltpu.async_copy(
            attn_hbm.at[batch, pl.ds(0, 64 * _S)],
            buf.at[pl.ds(0, 64 * _S)],
            sem1,
        )
        copy2 = pltpu.async_copy(
            attn_hbm.at[batch, pl.ds(64 * _S, 63 * _S)],
            buf.at[pl.ds(64 * _S, 63 * _S)],
            sem2,
        )

        lane = lax.iota(jnp.int32, 16)

        def sel(a, b):
            return jnp.where(half == 0, a, b)

        d0s = [sel(1, 33), sel(17, 49), sel(97, 65), sel(113, 81)]
        dvs = [d0 + lane for d0 in d0s]
        nchunk_seq = [4, 3, 2, 2, 1]
        nblocks = [sel(8, 24), 8, sel(16, 0), sel(24, 8), 8]

        def unit(base, dv, i, sacc, qacc):
            x = buf[pl.ds(base, 16)]
            m = (dv + i) < _S
            x = jnp.where(m, x, 0.0)
            return sacc + x, qacc + x * x

        zero = jnp.zeros((16,), jnp.float32)
        accs = [(zero, zero)] * 4
        row0 = 0
        copy1.wait()
        for phase in range(5):
            if phase == 3:
                copy2.wait()
            nchunks = nchunk_seq[phase]

            def body(j, carry, row0=row0, nchunks=nchunks):
                out = list(carry)
                i = row0 + j * 2
                for k in range(nchunks):
                    sc_, qc_ = out[2 * k], out[2 * k + 1]
                    base = d0s[k] + 129 * i
                    sc_, qc_ = unit(base, dvs[k], i, sc_, qc_)
                    sc_, qc_ = unit(base + 129, dvs[k], i + 1, sc_, qc_)
                    out[2 * k], out[2 * k + 1] = sc_, qc_
                return tuple(out)

            flat_accs = tuple(x for pair in accs[:nchunks] for x in pair)
            flat_accs = lax.fori_loop(0, nblocks[phase], body, flat_accs)
            for k in range(nchunks):
                accs[k] = (flat_accs[2 * k], flat_accs[2 * k + 1])
            row0 = row0 + nblocks[phase] * 2

        def finalize(dv, sx, qx):
            nf = (_S - dv).astype(jnp.float32)
            var = (qx - sx * sx / nf) / (nf - 1.0)
            var = jnp.maximum(var, 0.0)
            std = _sqrt16(var)
            return jnp.where(dv <= _S - 2, std * nf * 0.2, 0.0)

        partial = zero
        for k in range(4):
            partial = partial + finalize(dvs[k], accs[k][0], accs[k][1])
        part_v[...] = partial * _INV_COUNT
        pltpu.sync_copy(part_v, out_hbm.at[s])

    return diag_std_kernel


_diag_std = _make_kernel()


def kernel(attn):
    flat = attn.reshape(_B, _FLAT)
    out = _diag_std(flat)
    return jnp.sum(out)


# speedup vs baseline: 1.0772x; 1.0416x over previous
import functools

import jax
import jax.numpy as jnp
from jax import lax
from jax.experimental import pallas as pl
from jax.experimental.pallas import tpu as pltpu
from jax.experimental.pallas import tpu_sc as plsc

_S = 128
_B = 8
_FLAT = _S * _S
_PAD = 64
_INV_COUNT = 1.0 / (_B * (_S - 2))


def _make_kernel():
    mesh = plsc.VectorSubcoreMesh(
        core_axis_name="c", subcore_axis_name="s", num_cores=1
    )

    @functools.partial(
        pl.kernel,
        mesh=mesh,
        out_type=jax.ShapeDtypeStruct((16, 16), jnp.float32),
        compiler_params=pltpu.CompilerParams(needs_layout_passes=False),
        scratch_types=[
            pltpu.VMEM((_FLAT + _PAD,), jnp.float32),
            pltpu.VMEM((16,), jnp.float32),
        ],
    )
    def diag_std_kernel(attn_hbm, out_hbm, buf, part_v):
        s = lax.axis_index("s")
        batch = s >> 1
        half = s & 1

        def _copy(nrows):
            def f():
                pltpu.sync_copy(
                    attn_hbm.at[batch, pl.ds(0, nrows * _S)],
                    buf.at[pl.ds(0, nrows * _S)],
                )
            return f

        lax.switch(half, [_copy(127), _copy(95)])

        lane = lax.iota(jnp.int32, 16)

        def sel(a, b):
            return jnp.where(half == 0, a, b)

        d0s = [sel(1, 33), sel(17, 49), sel(97, 65), sel(113, 81)]
        dvs = [d0 + lane for d0 in d0s]
        nblocks = [sel(8, 24), 8, sel(40, 8), 8]

        def unit(base, dv, i, sacc, qacc):
            x = buf[pl.ds(base, 16)]
            m = (dv + i) < _S
            x = jnp.where(m, x, 0.0)
            return sacc + x, qacc + x * x

        zero = jnp.zeros((16,), jnp.float32)
        accs = [(zero, zero)] * 4
        row0 = 0
        for phase in range(4):
            nchunks = 4 - phase

            def body(j, carry, row0=row0, nchunks=nchunks):
                out = list(carry)
                i = row0 + j * 2
                for k in range(nchunks):
                    sc_, qc_ = out[2 * k], out[2 * k + 1]
                    base = d0s[k] + 129 * i
                    sc_, qc_ = unit(base, dvs[k], i, sc_, qc_)
                    sc_, qc_ = unit(base + 129, dvs[k], i + 1, sc_, qc_)
                    out[2 * k], out[2 * k + 1] = sc_, qc_
                return tuple(out)

            flat_accs = tuple(x for pair in accs[:nchunks] for x in pair)
            flat_accs = lax.fori_loop(0, nblocks[phase], body, flat_accs)
            for k in range(nchunks):
                accs[k] = (flat_accs[2 * k], flat_accs[2 * k + 1])
            row0 = row0 + nblocks[phase] * 2

        nfs = [(_S - dv).astype(jnp.float32) for dv in dvs]
        var4 = [
            jnp.maximum((q - s_ * s_ / nf) / (nf - 1.0), 0.0)
            for (s_, q), nf in zip(accs, nfs)
        ]
        ys = tuple((v + 1.0) * 0.5 for v in var4)

        def newton(_, ys):
            return tuple(0.5 * (y + v / y) for y, v in zip(ys, var4))

        ys = lax.fori_loop(0, 12, newton, ys)

        partial = zero
        for k in range(4):
            partial = partial + jnp.where(
                dvs[k] <= _S - 2, ys[k] * nfs[k] * 0.2, 0.0
            )
        part_v[...] = partial * _INV_COUNT
        pltpu.sync_copy(part_v, out_hbm.at[s])

    return diag_std_kernel


_diag_std = _make_kernel()


def kernel(attn):
    flat = attn.reshape(_B, _FLAT)
    out = _diag_std(flat)
    return jnp.sum(out)


# speedup vs baseline: 1.0817x; 1.0042x over previous
import functools

import jax
import jax.numpy as jnp
from jax import lax
from jax.experimental import pallas as pl
from jax.experimental.pallas import tpu as pltpu
from jax.experimental.pallas import tpu_sc as plsc

_S = 128
_B = 8
_FLAT = _S * _S
_PAD = 64
_INV_COUNT = 1.0 / (_B * (_S - 2))


def _make_kernel():
    mesh = plsc.VectorSubcoreMesh(
        core_axis_name="c", subcore_axis_name="s", num_cores=1
    )

    @functools.partial(
        pl.kernel,
        mesh=mesh,
        out_type=jax.ShapeDtypeStruct((16, 16), jnp.float32),
        compiler_params=pltpu.CompilerParams(needs_layout_passes=False),
        scratch_types=[
            pltpu.VMEM((_FLAT + _PAD,), jnp.float32),
            pltpu.VMEM((16,), jnp.float32),
        ],
    )
    def diag_std_kernel(attn_hbm, out_hbm, buf, part_v):
        s = lax.axis_index("s")
        batch = s >> 1
        half = s & 1

        def _copy(nrows):
            def f():
                pltpu.sync_copy(
                    attn_hbm.at[batch, pl.ds(0, nrows * _S)],
                    buf.at[pl.ds(0, nrows * _S)],
                )
            return f

        lax.switch(half, [_copy(127), _copy(95)])

        lane = lax.iota(jnp.int32, 16)

        def sel(a, b):
            return jnp.where(half == 0, a, b)

        d0s = [sel(1, 33), sel(17, 49), sel(97, 65), sel(113, 81)]
        dvs = [d0 + lane for d0 in d0s]
        nchunk_seq = [4, 2, 1]
        nblocks = [sel(16, 32), sel(40, 8), 8]

        def unit(base, dv, i, sacc, qacc):
            x = buf[pl.ds(base, 16)]
            m = (dv + i) < _S
            x = jnp.where(m, x, 0.0)
            return sacc + x, qacc + x * x

        zero = jnp.zeros((16,), jnp.float32)
        accs = [(zero, zero)] * 4
        row0 = 0
        for phase in range(3):
            nchunks = nchunk_seq[phase]

            def body(j, carry, row0=row0, nchunks=nchunks):
                out = list(carry)
                i = row0 + j * 2
                for k in range(nchunks):
                    sc_, qc_ = out[2 * k], out[2 * k + 1]
                    base = d0s[k] + 129 * i
                    sc_, qc_ = unit(base, dvs[k], i, sc_, qc_)
                    sc_, qc_ = unit(base + 129, dvs[k], i + 1, sc_, qc_)
                    out[2 * k], out[2 * k + 1] = sc_, qc_
                return tuple(out)

            flat_accs = tuple(x for pair in accs[:nchunks] for x in pair)
            flat_accs = lax.fori_loop(0, nblocks[phase], body, flat_accs)
            for k in range(nchunks):
                accs[k] = (flat_accs[2 * k], flat_accs[2 * k + 1])
            row0 = row0 + nblocks[phase] * 2

        nfs = [(_S - dv).astype(jnp.float32) for dv in dvs]
        var4 = [
            jnp.maximum((q - s_ * s_ / nf) / (nf - 1.0), 0.0)
            for (s_, q), nf in zip(accs, nfs)
        ]
        ys = tuple((v + 1.0) * 0.5 for v in var4)

        def newton(_, ys):
            return tuple(0.5 * (y + v / y) for y, v in zip(ys, var4))

        ys = lax.fori_loop(0, 12, newton, ys)

        partial = zero
        for k in range(4):
            partial = partial + jnp.where(
                dvs[k] <= _S - 2, ys[k] * nfs[k] * 0.2, 0.0
            )
        part_v[...] = partial * _INV_COUNT
        pltpu.sync_copy(part_v, out_hbm.at[s])

    return diag_std_kernel


_diag_std = _make_kernel()


def kernel(attn):
    flat = attn.reshape(_B, _FLAT)
    out = _diag_std(flat)
    return jnp.sum(out)
